# baseline (speedup 1.0000x reference)
import functools

import jax
import jax.numpy as jnp
from jax import lax
from jax.experimental import pallas as pl
from jax.experimental.pallas import tpu as pltpu
from jax.experimental.pallas import tpu_sc as plsc

N = 10000
E = 320000
B = 64
D = 128
H = 128
ED = 11

NC = 2
NS = 16
NW = NC * NS
C = 48
CHUNKS = 212
EW = CHUNKS * C
EPAD = EW * NW
NPAD = 10240
ROWS_PER_TILE = NPAD // NS
JV = H // 16

f32 = jnp.float32

_GDN = lax.GatherDimensionNumbers(
    offset_dims=(), collapsed_slice_dims=(0,), start_index_map=(0,))


def _take16(v, idx):
    return lax.gather(v, idx[:, None], _GDN, (1,),
                      mode=lax.GatherScatterMode.PROMISE_IN_BOUNDS)


def _paw_edge_body(h_hbm, he_hbm, src_hbm, dst_hbm, a_hbm,
                   num_hbm, den_hbm,
                   si0, si1, si2, si3, di0, di1, di2, di3,
                   hs0, hs1, hd0, hd1, he0, he1,
                   ex0, ex1, a_s,
                   den_sh, num_sh,
                   gsem0, gsem1, snum0, snum1, sden0, sden1,
                   isem0, isem1, isem2, isem3):
    c = lax.axis_index("c")
    s = lax.axis_index("s")
    wid = s * NC + c
    sidx, didx = (si0, si1, si2, si3), (di0, di1, di2, di3)
    isem = (isem0, isem1, isem2, isem3)
    hsb, hdb, heb = (hs0, hs1), (hd0, hd1), (he0, he1)
    exb = (ex0, ex1)
    gsem, snum, sden = (gsem0, gsem1), (snum0, snum1), (sden0, sden1)

    def zhs(i, carry):
        for j in range(JV):
            hs0[i, pl.ds(16 * j, 16)] = jnp.zeros((16,), f32)
        return carry
    lax.fori_loop(0, C, zhs, 0)

    def zex(i, carry):
        ex0[pl.ds(i * 16, 16)] = jnp.zeros((16,), f32)
        return carry
    lax.fori_loop(0, C // 16, zex, 0)

    row0 = s * ROWS_PER_TILE
    off = 0
    szs = [C] * (ROWS_PER_TILE // C) + ([ROWS_PER_TILE % C]
                                        if ROWS_PER_TILE % C else [])
    for sz in szs:
        pltpu.sync_copy(hs0.at[pl.ds(0, sz)], num_sh.at[pl.ds(row0 + off, sz)])
        pltpu.sync_copy(ex0.at[pl.ds(0, sz)], den_sh.at[pl.ds(row0 + off, sz)])
        off += sz

    pltpu.sync_copy(a_hbm, a_s)
    plsc.subcore_barrier()

    lane_iota = lax.iota(jnp.int32, 16)
    perms = [lane_iota ^ sh for sh in (1, 2, 4, 8)]
    av = [a_s[pl.ds(16 * j, 16)] for j in range(JV)]

    def fire_idx(k, q):
        base = wid * EW + k * C
        pltpu.async_copy(src_hbm.at[pl.ds(base, C)], sidx[q], isem[q])
        pltpu.async_copy(dst_hbm.at[pl.ds(base, C)], didx[q], isem[q])

    def wait_idx(q):
        pltpu.make_async_copy(src_hbm.at[pl.ds(0, C)], sidx[q], isem[q]).wait()
        pltpu.make_async_copy(dst_hbm.at[pl.ds(0, C)], didx[q], isem[q]).wait()

    def issue(k, q, b):
        base = wid * EW + k * C
        pltpu.async_copy(h_hbm.at[sidx[q]], hsb[b], gsem[b])
        pltpu.async_copy(h_hbm.at[didx[q]], hdb[b], gsem[b])
        pltpu.async_copy(he_hbm.at[pl.ds(base, C)], heb[b], gsem[b])

    def wait_gathers(b):
        pltpu.make_async_copy(h_hbm.at[sidx[0]], hsb[b], gsem[b]).wait()
        pltpu.make_async_copy(h_hbm.at[didx[0]], hdb[b], gsem[b]).wait()
        pltpu.make_async_copy(he_hbm.at[pl.ds(0, C)], heb[b], gsem[b]).wait()

    def scatter(q, b):
        pltpu.async_copy(hsb[b], num_sh.at[didx[q]], snum[b], add=True)
        pltpu.async_copy(exb[b], den_sh.at[didx[q]], sden[b], add=True)

    def wait_scatters(b):
        pltpu.make_async_copy(hsb[b], num_sh.at[didx[0]], snum[b]).wait()
        pltpu.make_async_copy(exb[b], den_sh.at[didx[0]], sden[b]).wait()

    def compute(b):
        hs, hd, he_s, ex_s = hsb[b], hdb[b], heb[b], exb[b]

        def group(g, gcarry):
            e0 = g * 16
            ex16 = jnp.zeros((16,), f32)
            for el in range(16):
                e = e0 + el
                acc = jnp.zeros((16,), f32)
                hsv = []
                for j in range(JV):
                    sl = pl.ds(16 * j, 16)
                    hj = hs[e, sl]
                    hsv.append(hj)
                    m = hj + hd[e, sl] + he_s[e, sl]
                    m = 0.6 * m + 0.4 * jnp.abs(m)
                    acc = acc + m * av[j]
                for p in perms:
                    acc = acc + _take16(acc, p)
                exv = jnp.exp(acc)
                ex16 = jnp.where(lane_iota == el, exv, ex16)
                for j in range(JV):
                    hs[e, pl.ds(16 * j, 16)] = hsv[j] * exv
            ex_s[pl.ds(e0, 16)] = ex16
            return gcarry
        lax.fori_loop(0, C // 16, group, 0)

    fire_idx(0, 0)
    fire_idx(1, 1)
    wait_idx(0)
    issue(0, 0, 0)

    def quad(kk, carry):
        for qb in range(4):
            k = kk * 4 + qb
            d = qb % 2
            @pl.when(k + 2 < CHUNKS)
            def _():
                fire_idx(k + 2, (qb + 2) % 4)
            wait_gathers(d)
            @pl.when(k >= 1)
            def _():
                wait_scatters(1 - d)

            @pl.when(k + 1 < CHUNKS)
            def _():
                wait_idx((qb + 1) % 4)
                issue(k + 1, (qb + 1) % 4, 1 - d)
            compute(d)
            scatter(qb, d)
        return carry
    lax.fori_loop(0, CHUNKS // 4, quad, 0)
    wait_scatters(1)

    plsc.subcore_barrier()
    pltpu.sync_copy(den_sh.at[pl.ds(row0, ROWS_PER_TILE)],
                    den_hbm.at[c, pl.ds(row0, ROWS_PER_TILE)])
    pltpu.sync_copy(num_sh.at[pl.ds(row0, ROWS_PER_TILE)],
                    num_hbm.at[c, pl.ds(row0, ROWS_PER_TILE)])


def _paw_edge_sc(h, he, src_p, dst_p, a):
    mesh = plsc.VectorSubcoreMesh(core_axis_name="c", subcore_axis_name="s")
    kfn = functools.partial(
        pl.kernel,
        mesh=mesh,
        out_type=[jax.ShapeDtypeStruct((NC, NPAD, H), f32),
                  jax.ShapeDtypeStruct((NC, NPAD), f32)],
        scratch_types=(
            [pltpu.VMEM((C,), jnp.int32)] * 8 +
            [pltpu.VMEM((C, H), f32)] * 6 +
            [pltpu.VMEM((C,), f32)] * 2 +
            [pltpu.VMEM((H,), f32)] +
            [pltpu.VMEM_SHARED((NPAD,), f32),
             pltpu.VMEM_SHARED((NPAD, H), f32)] +
            [pltpu.SemaphoreType.DMA] * 10
        ),
    )(_paw_edge_body)
    return kfn(h, he, src_p, dst_p, a)


def _mm_body(x_ref, w_ref, o_ref):
    o_ref[...] = jnp.dot(x_ref[...], w_ref[...], preferred_element_type=f32)


def _mm(x, w, bm):
    M, K = x.shape
    _, Ho = w.shape
    return pl.pallas_call(
        _mm_body,
        grid=(M // bm,),
        in_specs=[pl.BlockSpec((bm, K), lambda i: (i, 0)),
                  pl.BlockSpec((K, Ho), lambda i: (0, 0))],
        out_specs=pl.BlockSpec((bm, Ho), lambda i: (i, 0)),
        out_shape=jax.ShapeDtypeStruct((M, Ho), f32),
    )(x, w)


def _combine_body(num_ref, den_ref, o_ref):
    ssum = num_ref[0] + num_ref[1]
    dsum = den_ref[0, 0, 0, :] + den_ref[1, 0, 0, :] + 1e-16
    z = ssum / dsum[:, None]
    o_ref[...] = jnp.where(z > 0, z, jnp.exp(jnp.minimum(z, 0.0)) - 1.0)


def _combine_elu(num, den, bm):
    den4 = den[:, :N].reshape(NC, N // bm, 1, bm)
    return pl.pallas_call(
        _combine_body,
        grid=(N // bm,),
        in_specs=[pl.BlockSpec((NC, bm, H), lambda i: (0, i, 0)),
                  pl.BlockSpec((NC, 1, 1, bm), lambda i: (0, i, 0, 0))],
        out_specs=pl.BlockSpec((bm, H), lambda i: (i, 0)),
        out_shape=jax.ShapeDtypeStruct((N, H), f32),
    )(num, den4)


_BM = 400


def _metal_body(mf, mW1, mb1, mW2, mb2, Wq, bq, metal_o, qp_o):
    m = jnp.maximum(jnp.dot(mf[...], mW1[...], preferred_element_type=f32)
                    + mb1[...], 0.0)
    metal = jnp.dot(m, mW2[...], preferred_element_type=f32) + mb2[...]
    metal_o[...] = metal
    qp_o[...] = jnp.dot(metal, Wq[...], preferred_element_type=f32) + bq[...]


def _scores_body(h_ref, bat_ref, Wk, bk, qp, o_scores, o_smax):
    i = pl.program_id(0)
    kp = jnp.dot(h_ref[...], Wk[...], preferred_element_type=f32) + bk[...]
    bb = bat_ref[0, 0, :]
    oh = (bb[:, None] == lax.broadcasted_iota(jnp.int32, (1, B), 1)
          ).astype(f32)
    qb = jnp.dot(oh, qp[...], preferred_element_type=f32)
    sc = jnp.sum(qb * kp, axis=1) / jnp.sqrt(jnp.float32(H))
    o_scores[0, 0, :] = sc
    maskT = bb[None, :] == lax.broadcasted_iota(jnp.int32, (B, 1), 0)
    part = jnp.max(jnp.where(maskT, sc[None, :], -jnp.inf), axis=1)
    prev = jnp.where(i == 0, jnp.full((1, B), -jnp.inf, f32), o_smax[...])
    o_smax[...] = jnp.maximum(prev, part[None, :])


def _attnsum_body(h_ref, bat_ref, sc_ref, smax_ref, Wv, bv, o_num, o_den):
    i = pl.program_id(0)
    vp = jnp.dot(h_ref[...], Wv[...], preferred_element_type=f32) + bv[...]
    sm = smax_ref[0, :]
    sm = jnp.where(jnp.isfinite(sm), sm, 0.0)
    bb = bat_ref[0, 0, :]
    oh = (bb[:, None] == lax.broadcasted_iota(jnp.int32, (1, B), 1)
          ).astype(f32)
    smg = jnp.dot(oh, sm[:, None], preferred_element_type=f32)[:, 0]
    ex = jnp.exp(sc_ref[0, 0, :] - smg)
    ohT = (bb[None, :] == lax.broadcasted_iota(jnp.int32, (B, 1), 0)
           ).astype(f32)
    num_part = jnp.dot(ohT, ex[:, None] * vp, preferred_element_type=f32)
    den_part = jnp.dot(ohT, ex[:, None], preferred_element_type=f32)
    pn = jnp.where(i == 0, jnp.zeros((B, H), f32), o_num[...])
    pd = jnp.where(i == 0, jnp.zeros((B, 1), f32), o_den[...])
    o_num[...] = pn + num_part
    o_den[...] = pd + den_part


def _final_body(num, den, metal, w0, b0, w1, b1, w2, b2, w3, b3, w4, b4, out):
    attn = num[...] / (den[...] + 1e-16)
    z = jnp.concatenate([attn, metal[...]], axis=1)
    z = jnp.maximum(jnp.dot(z, w0[...], preferred_element_type=f32) + b0[...], 0.0)
    z = jnp.maximum(jnp.dot(z, w1[...], preferred_element_type=f32) + b1[...], 0.0)
    z = jnp.maximum(jnp.dot(z, w2[...], preferred_element_type=f32) + b2[...], 0.0)
    z = jnp.maximum(jnp.dot(z, w3[...], preferred_element_type=f32) + b3[...], 0.0)
    out[...] = jnp.dot(z, w4[...], preferred_element_type=f32) + b4[...]


def kernel(x, edge_index, edge_attr, batch, metal_features, params):
    src = edge_index[0]
    dst = edge_index[1]
    src_p = jnp.pad(src, (0, EPAD - E))
    dst_p = jnp.pad(dst, (0, EPAD - E), constant_values=N)
    ea_p = jnp.pad(edge_attr, ((0, EPAD - E), (0, 16 - ED)))

    def paw(f, W, We_p, a):
        h = _mm(f, W, _BM)
        he = _mm(ea_p, We_p, 2544)
        num, den = _paw_edge_sc(h, he, src_p, dst_p, a)
        return _combine_elu(num, den, _BM)

    pad_We = lambda We: jnp.pad(We, ((0, 16 - ED), (0, 0)))

    f = x
    for W, We, a in zip(params['node_W'], params['node_We'], params['node_a']):
        f = paw(f, W, pad_We(We), a)
    hcat = jnp.concatenate([x, f], axis=1)
    for W, We, a in zip(params['gat_W'], params['gat_We'], params['gat_a']):
        hcat = paw(hcat, W, pad_We(We), a)

    rb1 = lambda b: b.reshape(1, -1)
    metal, qp = pl.pallas_call(
        _metal_body,
        out_shape=[jax.ShapeDtypeStruct((B, H), f32),
                   jax.ShapeDtypeStruct((B, H), f32)],
    )(metal_features, params['mW1'], rb1(params['mb1']),
      params['mW2'], rb1(params['mb2']), params['Wq'], rb1(params['bq']))

    bat3 = batch.reshape(N // _BM, 1, _BM)
    scores, smax = pl.pallas_call(
        _scores_body,
        grid=(N // _BM,),
        in_specs=[pl.BlockSpec((_BM, H), lambda i: (i, 0)),
                  pl.BlockSpec((1, 1, _BM), lambda i: (i, 0, 0)),
                  pl.BlockSpec((H, H), lambda i: (0, 0)),
                  pl.BlockSpec((1, H), lambda i: (0, 0)),
                  pl.BlockSpec((B, H), lambda i: (0, 0))],
        out_specs=[pl.BlockSpec((1, 1, _BM), lambda i: (i, 0, 0)),
                   pl.BlockSpec((1, B), lambda i: (0, 0))],
        out_shape=[jax.ShapeDtypeStruct((N // _BM, 1, _BM), f32),
                   jax.ShapeDtypeStruct((1, B), f32)],
    )(hcat, bat3, params['Wk'], rb1(params['bk']), qp)

    anum, aden = pl.pallas_call(
        _attnsum_body,
        grid=(N // _BM,),
        in_specs=[pl.BlockSpec((_BM, H), lambda i: (i, 0)),
                  pl.BlockSpec((1, 1, _BM), lambda i: (i, 0, 0)),
                  pl.BlockSpec((1, 1, _BM), lambda i: (i, 0, 0)),
                  pl.BlockSpec((1, B), lambda i: (0, 0)),
                  pl.BlockSpec((H, H), lambda i: (0, 0)),
                  pl.BlockSpec((1, H), lambda i: (0, 0))],
        out_specs=[pl.BlockSpec((B, H), lambda i: (0, 0)),
                   pl.BlockSpec((B, 1), lambda i: (0, 0))],
        out_shape=[jax.ShapeDtypeStruct((B, H), f32),
                   jax.ShapeDtypeStruct((B, 1), f32)],
    )(hcat, bat3, scores, smax, params['Wv'], rb1(params['bv']))

    rW = params['reg_W']
    rb = [rb1(b) for b in params['reg_b']]
    out = pl.pallas_call(
        _final_body,
        out_shape=jax.ShapeDtypeStruct((B, 1), f32),
    )(anum, aden, metal,
      rW[0], rb[0], rW[1], rb[1], rW[2], rb[2], rW[3], rb[3], rW[4], rb[4])
    return jnp.squeeze(out, -1)


# speedup vs baseline: 1.0158x; 1.0158x over previous
import functools

import jax
import jax.numpy as jnp
from jax import lax
from jax.experimental import pallas as pl
from jax.experimental.pallas import tpu as pltpu
from jax.experimental.pallas import tpu_sc as plsc

N = 10000
E = 320000
B = 64
D = 128
H = 128
ED = 11

NC = 2
NS = 16
NW = NC * NS
C = 48
CHUNKS = 210
EW = CHUNKS * C
EPAD = EW * NW
NPAD = 10240
ROWS_PER_TILE = NPAD // NS
JV = H // 16

f32 = jnp.float32

_GDN = lax.GatherDimensionNumbers(
    offset_dims=(), collapsed_slice_dims=(0,), start_index_map=(0,))


def _take16(v, idx):
    return lax.gather(v, idx[:, None], _GDN, (1,),
                      mode=lax.GatherScatterMode.PROMISE_IN_BOUNDS)


def _paw_edge_body(h_hbm, he_hbm, src_hbm, dst_hbm, a_hbm,
                   num_hbm, den_hbm,
                   src0, src1, dst0, dst1, hs0, hs1, hd0, hd1, he0, he1,
                   ex0, ex1, a_s,
                   den_sh, num_sh,
                   gsem0, gsem1, snum0, snum1, sden0, sden1):
    c = lax.axis_index("c")
    s = lax.axis_index("s")
    wid = s * NC + c
    srcb, dstb = (src0, src1), (dst0, dst1)
    hsb, hdb, heb = (hs0, hs1), (hd0, hd1), (he0, he1)
    exb = (ex0, ex1)
    gsem, snum, sden = (gsem0, gsem1), (snum0, snum1), (sden0, sden1)

    def zhs(i, carry):
        for j in range(JV):
            hs0[i, pl.ds(16 * j, 16)] = jnp.zeros((16,), f32)
        return carry
    lax.fori_loop(0, C, zhs, 0)

    def zex(i, carry):
        ex0[pl.ds(i * 16, 16)] = jnp.zeros((16,), f32)
        return carry
    lax.fori_loop(0, C // 16, zex, 0)

    row0 = s * ROWS_PER_TILE
    off = 0
    szs = [C] * (ROWS_PER_TILE // C) + ([ROWS_PER_TILE % C]
                                        if ROWS_PER_TILE % C else [])
    for sz in szs:
        pltpu.sync_copy(hs0.at[pl.ds(0, sz)], num_sh.at[pl.ds(row0 + off, sz)])
        pltpu.sync_copy(ex0.at[pl.ds(0, sz)], den_sh.at[pl.ds(row0 + off, sz)])
        off += sz

    pltpu.sync_copy(a_hbm, a_s)
    plsc.subcore_barrier()

    lane_iota = lax.iota(jnp.int32, 16)
    perms = [lane_iota ^ sh for sh in (1, 2, 4, 8)]
    av = [a_s[pl.ds(16 * j, 16)] for j in range(JV)]

    def issue(k, b):
        base = wid * EW + k * C
        ca = pltpu.async_copy(src_hbm.at[pl.ds(base, C)], srcb[b], gsem[b])
        cb = pltpu.async_copy(dst_hbm.at[pl.ds(base, C)], dstb[b], gsem[b])
        ca.wait()
        cb.wait()
        pltpu.async_copy(h_hbm.at[srcb[b]], hsb[b], gsem[b])
        pltpu.async_copy(h_hbm.at[dstb[b]], hdb[b], gsem[b])
        pltpu.async_copy(he_hbm.at[pl.ds(base, C)], heb[b], gsem[b])

    def wait_gathers(b):
        pltpu.make_async_copy(h_hbm.at[srcb[b]], hsb[b], gsem[b]).wait()
        pltpu.make_async_copy(h_hbm.at[dstb[b]], hdb[b], gsem[b]).wait()
        pltpu.make_async_copy(he_hbm.at[pl.ds(0, C)], heb[b], gsem[b]).wait()

    def wait_scatters(b):
        pltpu.make_async_copy(hsb[b], num_sh.at[dstb[b]], snum[b]).wait()
        pltpu.make_async_copy(exb[b], den_sh.at[dstb[b]], sden[b]).wait()

    def compute(b):
        hs, hd, he_s, ex_s = hsb[b], hdb[b], heb[b], exb[b]

        def group(g, gcarry):
            e0 = g * 16
            ex16 = jnp.zeros((16,), f32)
            for el in range(16):
                e = e0 + el
                acc = jnp.zeros((16,), f32)
                hsv = []
                for j in range(JV):
                    sl = pl.ds(16 * j, 16)
                    hj = hs[e, sl]
                    hsv.append(hj)
                    m = hj + hd[e, sl] + he_s[e, sl]
                    m = jnp.maximum(m, 0.2 * m)
                    acc = acc + m * av[j]
                for p in perms:
                    acc = acc + _take16(acc, p)
                exv = jnp.exp(acc)
                ex16 = jnp.where(lane_iota == el, exv, ex16)
                for j in range(JV):
                    hs[e, pl.ds(16 * j, 16)] = hsv[j] * exv
            ex_s[pl.ds(e0, 16)] = ex16
            return gcarry
        lax.fori_loop(0, C // 16, group, 0)

    issue(0, 0)

    def pair(kk, carry):
        for b in range(2):
            k = kk * 2 + b
            wait_gathers(b)
            @pl.when(k >= 1)
            def _():
                wait_scatters(1 - b)

            @pl.when(k + 1 < CHUNKS)
            def _():
                issue(k + 1, 1 - b)
            compute(b)
            pltpu.async_copy(hsb[b], num_sh.at[dstb[b]], snum[b], add=True)
            pltpu.async_copy(exb[b], den_sh.at[dstb[b]], sden[b], add=True)
        return carry
    lax.fori_loop(0, CHUNKS // 2, pair, 0)
    wait_scatters(1)

    plsc.subcore_barrier()
    pltpu.sync_copy(den_sh.at[pl.ds(row0, ROWS_PER_TILE)],
                    den_hbm.at[c, pl.ds(row0, ROWS_PER_TILE)])
    pltpu.sync_copy(num_sh.at[pl.ds(row0, ROWS_PER_TILE)],
                    num_hbm.at[c, pl.ds(row0, ROWS_PER_TILE)])


def _paw_edge_sc(h, he, src_p, dst_p, a):
    mesh = plsc.VectorSubcoreMesh(core_axis_name="c", subcore_axis_name="s")
    kfn = functools.partial(
        pl.kernel,
        mesh=mesh,
        out_type=[jax.ShapeDtypeStruct((NC, NPAD, H), f32),
                  jax.ShapeDtypeStruct((NC, NPAD), f32)],
        scratch_types=(
            [pltpu.VMEM((C,), jnp.int32)] * 4 +
            [pltpu.VMEM((C, H), f32)] * 6 +
            [pltpu.VMEM((C,), f32)] * 2 +
            [pltpu.VMEM((H,), f32)] +
            [pltpu.VMEM_SHARED((NPAD,), f32),
             pltpu.VMEM_SHARED((NPAD, H), f32)] +
            [pltpu.SemaphoreType.DMA] * 6
        ),
    )(_paw_edge_body)
    return kfn(h, he, src_p, dst_p, a)


def _mm_body(x_ref, w_ref, o_ref):
    o_ref[...] = jnp.dot(x_ref[...], w_ref[...], preferred_element_type=f32)


def _mm(x, w, bm):
    M, K = x.shape
    _, Ho = w.shape
    return pl.pallas_call(
        _mm_body,
        grid=(M // bm,),
        in_specs=[pl.BlockSpec((bm, K), lambda i: (i, 0)),
                  pl.BlockSpec((K, Ho), lambda i: (0, 0))],
        out_specs=pl.BlockSpec((bm, Ho), lambda i: (i, 0)),
        out_shape=jax.ShapeDtypeStruct((M, Ho), f32),
    )(x, w)


def _combine_body(num_ref, den_ref, o_ref):
    ssum = num_ref[0] + num_ref[1]
    dsum = den_ref[0, 0, 0, :] + den_ref[1, 0, 0, :] + 1e-16
    z = ssum / dsum[:, None]
    o_ref[...] = jnp.where(z > 0, z, jnp.exp(jnp.minimum(z, 0.0)) - 1.0)


def _combine_elu(num, den, bm):
    den4 = den[:, :N].reshape(NC, N // bm, 1, bm)
    return pl.pallas_call(
        _combine_body,
        grid=(N // bm,),
        in_specs=[pl.BlockSpec((NC, bm, H), lambda i: (0, i, 0)),
                  pl.BlockSpec((NC, 1, 1, bm), lambda i: (0, i, 0, 0))],
        out_specs=pl.BlockSpec((bm, H), lambda i: (i, 0)),
        out_shape=jax.ShapeDtypeStruct((N, H), f32),
    )(num, den4)


_BM = 400


def _metal_body(mf, mW1, mb1, mW2, mb2, Wq, bq, metal_o, qp_o):
    m = jnp.maximum(jnp.dot(mf[...], mW1[...], preferred_element_type=f32)
                    + mb1[...], 0.0)
    metal = jnp.dot(m, mW2[...], preferred_element_type=f32) + mb2[...]
    metal_o[...] = metal
    qp_o[...] = jnp.dot(metal, Wq[...], preferred_element_type=f32) + bq[...]


def _scores_body(h_ref, bat_ref, Wk, bk, qp, o_scores, o_smax):
    i = pl.program_id(0)
    kp = jnp.dot(h_ref[...], Wk[...], preferred_element_type=f32) + bk[...]
    bb = bat_ref[0, 0, :]
    oh = (bb[:, None] == lax.broadcasted_iota(jnp.int32, (1, B), 1)
          ).astype(f32)
    qb = jnp.dot(oh, qp[...], preferred_element_type=f32)
    sc = jnp.sum(qb * kp, axis=1) / jnp.sqrt(jnp.float32(H))
    o_scores[0, 0, :] = sc
    maskT = bb[None, :] == lax.broadcasted_iota(jnp.int32, (B, 1), 0)
    part = jnp.max(jnp.where(maskT, sc[None, :], -jnp.inf), axis=1)
    prev = jnp.where(i == 0, jnp.full((1, B), -jnp.inf, f32), o_smax[...])
    o_smax[...] = jnp.maximum(prev, part[None, :])


def _attnsum_body(h_ref, bat_ref, sc_ref, smax_ref, Wv, bv, o_num, o_den):
    i = pl.program_id(0)
    vp = jnp.dot(h_ref[...], Wv[...], preferred_element_type=f32) + bv[...]
    sm = smax_ref[0, :]
    sm = jnp.where(jnp.isfinite(sm), sm, 0.0)
    bb = bat_ref[0, 0, :]
    oh = (bb[:, None] == lax.broadcasted_iota(jnp.int32, (1, B), 1)
          ).astype(f32)
    smg = jnp.dot(oh, sm[:, None], preferred_element_type=f32)[:, 0]
    ex = jnp.exp(sc_ref[0, 0, :] - smg)
    ohT = (bb[None, :] == lax.broadcasted_iota(jnp.int32, (B, 1), 0)
           ).astype(f32)
    num_part = jnp.dot(ohT, ex[:, None] * vp, preferred_element_type=f32)
    den_part = jnp.dot(ohT, ex[:, None], preferred_element_type=f32)
    pn = jnp.where(i == 0, jnp.zeros((B, H), f32), o_num[...])
    pd = jnp.where(i == 0, jnp.zeros((B, 1), f32), o_den[...])
    o_num[...] = pn + num_part
    o_den[...] = pd + den_part


def _final_body(num, den, metal, w0, b0, w1, b1, w2, b2, w3, b3, w4, b4, out):
    attn = num[...] / (den[...] + 1e-16)
    z = jnp.concatenate([attn, metal[...]], axis=1)
    z = jnp.maximum(jnp.dot(z, w0[...], preferred_element_type=f32) + b0[...], 0.0)
    z = jnp.maximum(jnp.dot(z, w1[...], preferred_element_type=f32) + b1[...], 0.0)
    z = jnp.maximum(jnp.dot(z, w2[...], preferred_element_type=f32) + b2[...], 0.0)
    z = jnp.maximum(jnp.dot(z, w3[...], preferred_element_type=f32) + b3[...], 0.0)
    out[...] = jnp.dot(z, w4[...], preferred_element_type=f32) + b4[...]


def kernel(x, edge_index, edge_attr, batch, metal_features, params):
    src = edge_index[0]
    dst = edge_index[1]
    src_p = jnp.pad(src, (0, EPAD - E))
    dst_p = jnp.pad(dst, (0, EPAD - E), constant_values=N)
    ea_p = jnp.pad(edge_attr, ((0, EPAD - E), (0, 16 - ED)))

    def paw(f, W, We_p, a):
        h = _mm(f, W, _BM)
        he = _mm(ea_p, We_p, 2520)
        num, den = _paw_edge_sc(h, he, src_p, dst_p, a)
        return _combine_elu(num, den, _BM)

    pad_We = lambda We: jnp.pad(We, ((0, 16 - ED), (0, 0)))

    f = x
    for W, We, a in zip(params['node_W'], params['node_We'], params['node_a']):
        f = paw(f, W, pad_We(We), a)
    hcat = jnp.concatenate([x, f], axis=1)
    for W, We, a in zip(params['gat_W'], params['gat_We'], params['gat_a']):
        hcat = paw(hcat, W, pad_We(We), a)

    rb1 = lambda b: b.reshape(1, -1)
    metal, qp = pl.pallas_call(
        _metal_body,
        out_shape=[jax.ShapeDtypeStruct((B, H), f32),
                   jax.ShapeDtypeStruct((B, H), f32)],
    )(metal_features, params['mW1'], rb1(params['mb1']),
      params['mW2'], rb1(params['mb2']), params['Wq'], rb1(params['bq']))

    bat3 = batch.reshape(N // _BM, 1, _BM)
    scores, smax = pl.pallas_call(
        _scores_body,
        grid=(N // _BM,),
        in_specs=[pl.BlockSpec((_BM, H), lambda i: (i, 0)),
                  pl.BlockSpec((1, 1, _BM), lambda i: (i, 0, 0)),
                  pl.BlockSpec((H, H), lambda i: (0, 0)),
                  pl.BlockSpec((1, H), lambda i: (0, 0)),
                  pl.BlockSpec((B, H), lambda i: (0, 0))],
        out_specs=[pl.BlockSpec((1, 1, _BM), lambda i: (i, 0, 0)),
                   pl.BlockSpec((1, B), lambda i: (0, 0))],
        out_shape=[jax.ShapeDtypeStruct((N // _BM, 1, _BM), f32),
                   jax.ShapeDtypeStruct((1, B), f32)],
    )(hcat, bat3, params['Wk'], rb1(params['bk']), qp)

    anum, aden = pl.pallas_call(
        _attnsum_body,
        grid=(N // _BM,),
        in_specs=[pl.BlockSpec((_BM, H), lambda i: (i, 0)),
                  pl.BlockSpec((1, 1, _BM), lambda i: (i, 0, 0)),
                  pl.BlockSpec((1, 1, _BM), lambda i: (i, 0, 0)),
                  pl.BlockSpec((1, B), lambda i: (0, 0)),
                  pl.BlockSpec((H, H), lambda i: (0, 0)),
                  pl.BlockSpec((1, H), lambda i: (0, 0))],
        out_specs=[pl.BlockSpec((B, H), lambda i: (0, 0)),
                   pl.BlockSpec((B, 1), lambda i: (0, 0))],
        out_shape=[jax.ShapeDtypeStruct((B, H), f32),
                   jax.ShapeDtypeStruct((B, 1), f32)],
    )(hcat, bat3, scores, smax, params['Wv'], rb1(params['bv']))

    rW = params['reg_W']
    rb = [rb1(b) for b in params['reg_b']]
    out = pl.pallas_call(
        _final_body,
        out_shape=jax.ShapeDtypeStruct((B, 1), f32),
    )(anum, aden, metal,
      rW[0], rb[0], rW[1], rb[1], rW[2], rb[2], rW[3], rb[3], rW[4], rb[4])
    return jnp.squeeze(out, -1)


# speedup vs baseline: 1.0855x; 1.0686x over previous
import functools

import jax
import jax.numpy as jnp
from jax import lax
from jax.experimental import pallas as pl
from jax.experimental.pallas import tpu as pltpu
from jax.experimental.pallas import tpu_sc as plsc

N = 10000
E = 320000
B = 64
D = 128
H = 128
ED = 11

NC = 2
NS = 16
NW = NC * NS
C = 48
CHUNKS = 210
EW = CHUNKS * C
EPAD = EW * NW
NPAD = 10240
ROWS_PER_TILE = NPAD // NS
JV = H // 16

f32 = jnp.float32

_GDN = lax.GatherDimensionNumbers(
    offset_dims=(), collapsed_slice_dims=(0,), start_index_map=(0,))


def _take16(v, idx):
    return lax.gather(v, idx[:, None], _GDN, (1,),
                      mode=lax.GatherScatterMode.PROMISE_IN_BOUNDS)


def _paw_edge_body(h_hbm, he_hbm, src_hbm, dst_hbm, a_hbm,
                   num_hbm, den_hbm,
                   src0, src1, dst0, dst1, hs0, hs1, hd0, hd1, he0, he1,
                   ex0, ex1, a_s,
                   den_sh, num_sh,
                   gsem0, gsem1, snum0, snum1, sden0, sden1):
    c = lax.axis_index("c")
    s = lax.axis_index("s")
    wid = s * NC + c
    srcb, dstb = (src0, src1), (dst0, dst1)
    hsb, hdb, heb = (hs0, hs1), (hd0, hd1), (he0, he1)
    exb = (ex0, ex1)
    gsem, snum, sden = (gsem0, gsem1), (snum0, snum1), (sden0, sden1)

    def zhs(i, carry):
        for j in range(JV):
            hs0[i, pl.ds(16 * j, 16)] = jnp.zeros((16,), f32)
        return carry
    lax.fori_loop(0, C, zhs, 0)

    def zex(i, carry):
        ex0[pl.ds(i * 16, 16)] = jnp.zeros((16,), f32)
        return carry
    lax.fori_loop(0, C // 16, zex, 0)

    row0 = s * ROWS_PER_TILE
    off = 0
    szs = [C] * (ROWS_PER_TILE // C) + ([ROWS_PER_TILE % C]
                                        if ROWS_PER_TILE % C else [])
    for sz in szs:
        pltpu.sync_copy(hs0.at[pl.ds(0, sz)], num_sh.at[pl.ds(row0 + off, sz)])
        pltpu.sync_copy(ex0.at[pl.ds(0, sz)], den_sh.at[pl.ds(row0 + off, sz)])
        off += sz

    pltpu.sync_copy(a_hbm, a_s)
    plsc.subcore_barrier()

    lane_iota = lax.iota(jnp.int32, 16)
    perms = [lane_iota ^ sh for sh in (1, 2, 4, 8)]
    av = [a_s[pl.ds(16 * j, 16)] for j in range(JV)]

    def issue(k, b):
        base = wid * EW + k * C
        ca = pltpu.async_copy(src_hbm.at[pl.ds(base, C)], srcb[b], gsem[b])
        cb = pltpu.async_copy(dst_hbm.at[pl.ds(base, C)], dstb[b], gsem[b])
        ca.wait()
        cb.wait()
        pltpu.async_copy(h_hbm.at[srcb[b]], hsb[b], gsem[b])
        pltpu.async_copy(h_hbm.at[dstb[b]], hdb[b], gsem[b])
        pltpu.async_copy(he_hbm.at[pl.ds(base, C)], heb[b], gsem[b])

    def wait_gathers(b):
        pltpu.make_async_copy(h_hbm.at[srcb[b]], hsb[b], gsem[b]).wait()
        pltpu.make_async_copy(h_hbm.at[dstb[b]], hdb[b], gsem[b]).wait()
        pltpu.make_async_copy(he_hbm.at[pl.ds(0, C)], heb[b], gsem[b]).wait()

    def wait_scatters(b):
        pltpu.make_async_copy(hsb[b], num_sh.at[dstb[b]], snum[b]).wait()
        pltpu.make_async_copy(exb[b], den_sh.at[dstb[b]], sden[b]).wait()

    def compute(b):
        hs, hd, he_s, ex_s = hsb[b], hdb[b], heb[b], exb[b]

        def group(g, gcarry):
            e0 = g * 16
            ex16 = jnp.zeros((16,), f32)
            for el in range(16):
                e = e0 + el
                acc = jnp.zeros((16,), f32)
                hsv = []
                for j in range(JV):
                    sl = pl.ds(16 * j, 16)
                    hj = hs[e, sl]
                    hsv.append(hj)
                    m = hj + hd[e, sl] + he_s[e, sl]
                    m = 0.6 * m + 0.4 * jnp.abs(m)
                    acc = acc + m * av[j]
                for p in perms:
                    acc = acc + _take16(acc, p)
                exv = jnp.exp(acc)
                ex16 = jnp.where(lane_iota == el, exv, ex16)
                for j in range(JV):
                    hs[e, pl.ds(16 * j, 16)] = hsv[j] * exv
            ex_s[pl.ds(e0, 16)] = ex16
            return gcarry
        lax.fori_loop(0, C // 16, group, 0)

    issue(0, 0)

    def pair(kk, carry):
        for b in range(2):
            k = kk * 2 + b
            wait_gathers(b)
            @pl.when(k >= 1)
            def _():
                wait_scatters(1 - b)

            @pl.when(k + 1 < CHUNKS)
            def _():
                issue(k + 1, 1 - b)
            compute(b)
            pltpu.async_copy(hsb[b], num_sh.at[dstb[b]], snum[b], add=True)
            pltpu.async_copy(exb[b], den_sh.at[dstb[b]], sden[b], add=True)
        return carry
    lax.fori_loop(0, CHUNKS // 2, pair, 0)
    wait_scatters(1)

    plsc.subcore_barrier()
    pltpu.sync_copy(den_sh.at[pl.ds(row0, ROWS_PER_TILE)],
                    den_hbm.at[c, pl.ds(row0, ROWS_PER_TILE)])
    pltpu.sync_copy(num_sh.at[pl.ds(row0, ROWS_PER_TILE)],
                    num_hbm.at[c, pl.ds(row0, ROWS_PER_TILE)])


def _paw_edge_sc(h, he, src_p, dst_p, a):
    mesh = plsc.VectorSubcoreMesh(core_axis_name="c", subcore_axis_name="s")
    kfn = functools.partial(
        pl.kernel,
        mesh=mesh,
        out_type=[jax.ShapeDtypeStruct((NC, NPAD, H), f32),
                  jax.ShapeDtypeStruct((NC, NPAD), f32)],
        scratch_types=(
            [pltpu.VMEM((C,), jnp.int32)] * 4 +
            [pltpu.VMEM((C, H), f32)] * 6 +
            [pltpu.VMEM((C,), f32)] * 2 +
            [pltpu.VMEM((H,), f32)] +
            [pltpu.VMEM_SHARED((NPAD,), f32),
             pltpu.VMEM_SHARED((NPAD, H), f32)] +
            [pltpu.SemaphoreType.DMA] * 6
        ),
    )(_paw_edge_body)
    return kfn(h, he, src_p, dst_p, a)


def _mm_body(x_ref, w_ref, o_ref):
    o_ref[...] = jnp.dot(x_ref[...], w_ref[...], preferred_element_type=f32)


def _mm(x, w, bm):
    M, K = x.shape
    _, Ho = w.shape
    return pl.pallas_call(
        _mm_body,
        grid=(M // bm,),
        in_specs=[pl.BlockSpec((bm, K), lambda i: (i, 0)),
                  pl.BlockSpec((K, Ho), lambda i: (0, 0))],
        out_specs=pl.BlockSpec((bm, Ho), lambda i: (i, 0)),
        out_shape=jax.ShapeDtypeStruct((M, Ho), f32),
    )(x, w)


def _combine_body(num_ref, den_ref, o_ref):
    ssum = num_ref[0] + num_ref[1]
    dsum = den_ref[0, 0, 0, :] + den_ref[1, 0, 0, :] + 1e-16
    z = ssum / dsum[:, None]
    o_ref[...] = jnp.where(z > 0, z, jnp.exp(jnp.minimum(z, 0.0)) - 1.0)


def _combine_elu(num, den, bm):
    den4 = den[:, :N].reshape(NC, N // bm, 1, bm)
    return pl.pallas_call(
        _combine_body,
        grid=(N // bm,),
        in_specs=[pl.BlockSpec((NC, bm, H), lambda i: (0, i, 0)),
                  pl.BlockSpec((NC, 1, 1, bm), lambda i: (0, i, 0, 0))],
        out_specs=pl.BlockSpec((bm, H), lambda i: (i, 0)),
        out_shape=jax.ShapeDtypeStruct((N, H), f32),
    )(num, den4)


_BM = 400


def _metal_body(mf, mW1, mb1, mW2, mb2, Wq, bq, metal_o, qp_o):
    m = jnp.maximum(jnp.dot(mf[...], mW1[...], preferred_element_type=f32)
                    + mb1[...], 0.0)
    metal = jnp.dot(m, mW2[...], preferred_element_type=f32) + mb2[...]
    metal_o[...] = metal
    qp_o[...] = jnp.dot(metal, Wq[...], preferred_element_type=f32) + bq[...]


def _scores_body(h_ref, bat_ref, Wk, bk, qp, o_scores, o_smax):
    i = pl.program_id(0)
    kp = jnp.dot(h_ref[...], Wk[...], preferred_element_type=f32) + bk[...]
    bb = bat_ref[0, 0, :]
    oh = (bb[:, None] == lax.broadcasted_iota(jnp.int32, (1, B), 1)
          ).astype(f32)
    qb = jnp.dot(oh, qp[...], preferred_element_type=f32)
    sc = jnp.sum(qb * kp, axis=1) / jnp.sqrt(jnp.float32(H))
    o_scores[0, 0, :] = sc
    maskT = bb[None, :] == lax.broadcasted_iota(jnp.int32, (B, 1), 0)
    part = jnp.max(jnp.where(maskT, sc[None, :], -jnp.inf), axis=1)
    prev = jnp.where(i == 0, jnp.full((1, B), -jnp.inf, f32), o_smax[...])
    o_smax[...] = jnp.maximum(prev, part[None, :])


def _attnsum_body(h_ref, bat_ref, sc_ref, smax_ref, Wv, bv, o_num, o_den):
    i = pl.program_id(0)
    vp = jnp.dot(h_ref[...], Wv[...], preferred_element_type=f32) + bv[...]
    sm = smax_ref[0, :]
    sm = jnp.where(jnp.isfinite(sm), sm, 0.0)
    bb = bat_ref[0, 0, :]
    oh = (bb[:, None] == lax.broadcasted_iota(jnp.int32, (1, B), 1)
          ).astype(f32)
    smg = jnp.dot(oh, sm[:, None], preferred_element_type=f32)[:, 0]
    ex = jnp.exp(sc_ref[0, 0, :] - smg)
    ohT = (bb[None, :] == lax.broadcasted_iota(jnp.int32, (B, 1), 0)
           ).astype(f32)
    num_part = jnp.dot(ohT, ex[:, None] * vp, preferred_element_type=f32)
    den_part = jnp.dot(ohT, ex[:, None], preferred_element_type=f32)
    pn = jnp.where(i == 0, jnp.zeros((B, H), f32), o_num[...])
    pd = jnp.where(i == 0, jnp.zeros((B, 1), f32), o_den[...])
    o_num[...] = pn + num_part
    o_den[...] = pd + den_part


def _final_body(num, den, metal, w0, b0, w1, b1, w2, b2, w3, b3, w4, b4, out):
    attn = num[...] / (den[...] + 1e-16)
    z = jnp.concatenate([attn, metal[...]], axis=1)
    z = jnp.maximum(jnp.dot(z, w0[...], preferred_element_type=f32) + b0[...], 0.0)
    z = jnp.maximum(jnp.dot(z, w1[...], preferred_element_type=f32) + b1[...], 0.0)
    z = jnp.maximum(jnp.dot(z, w2[...], preferred_element_type=f32) + b2[...], 0.0)
    z = jnp.maximum(jnp.dot(z, w3[...], preferred_element_type=f32) + b3[...], 0.0)
    out[...] = jnp.dot(z, w4[...], preferred_element_type=f32) + b4[...]


def kernel(x, edge_index, edge_attr, batch, metal_features, params):
    src = edge_index[0]
    dst = edge_index[1]
    src_p = jnp.pad(src, (0, EPAD - E))
    dst_p = jnp.pad(dst, (0, EPAD - E), constant_values=N)
    ea_p = jnp.pad(edge_attr, ((0, EPAD - E), (0, 16 - ED)))

    def paw(f, W, We_p, a):
        h = _mm(f, W, _BM)
        he = _mm(ea_p, We_p, 2520)
        num, den = _paw_edge_sc(h, he, src_p, dst_p, a)
        return _combine_elu(num, den, _BM)

    pad_We = lambda We: jnp.pad(We, ((0, 16 - ED), (0, 0)))

    f = x
    for W, We, a in zip(params['node_W'], params['node_We'], params['node_a']):
        f = paw(f, W, pad_We(We), a)
    hcat = jnp.concatenate([x, f], axis=1)
    for W, We, a in zip(params['gat_W'], params['gat_We'], params['gat_a']):
        hcat = paw(hcat, W, pad_We(We), a)

    rb1 = lambda b: b.reshape(1, -1)
    metal, qp = pl.pallas_call(
        _metal_body,
        out_shape=[jax.ShapeDtypeStruct((B, H), f32),
                   jax.ShapeDtypeStruct((B, H), f32)],
    )(metal_features, params['mW1'], rb1(params['mb1']),
      params['mW2'], rb1(params['mb2']), params['Wq'], rb1(params['bq']))

    bat3 = batch.reshape(N // _BM, 1, _BM)
    scores, smax = pl.pallas_call(
        _scores_body,
        grid=(N // _BM,),
        in_specs=[pl.BlockSpec((_BM, H), lambda i: (i, 0)),
                  pl.BlockSpec((1, 1, _BM), lambda i: (i, 0, 0)),
                  pl.BlockSpec((H, H), lambda i: (0, 0)),
                  pl.BlockSpec((1, H), lambda i: (0, 0)),
                  pl.BlockSpec((B, H), lambda i: (0, 0))],
        out_specs=[pl.BlockSpec((1, 1, _BM), lambda i: (i, 0, 0)),
                   pl.BlockSpec((1, B), lambda i: (0, 0))],
        out_shape=[jax.ShapeDtypeStruct((N // _BM, 1, _BM), f32),
                   jax.ShapeDtypeStruct((1, B), f32)],
    )(hcat, bat3, params['Wk'], rb1(params['bk']), qp)

    anum, aden = pl.pallas_call(
        _attnsum_body,
        grid=(N // _BM,),
        in_specs=[pl.BlockSpec((_BM, H), lambda i: (i, 0)),
                  pl.BlockSpec((1, 1, _BM), lambda i: (i, 0, 0)),
                  pl.BlockSpec((1, 1, _BM), lambda i: (i, 0, 0)),
                  pl.BlockSpec((1, B), lambda i: (0, 0)),
                  pl.BlockSpec((H, H), lambda i: (0, 0)),
                  pl.BlockSpec((1, H), lambda i: (0, 0))],
        out_specs=[pl.BlockSpec((B, H), lambda i: (0, 0)),
                   pl.BlockSpec((B, 1), lambda i: (0, 0))],
        out_shape=[jax.ShapeDtypeStruct((B, H), f32),
                   jax.ShapeDtypeStruct((B, 1), f32)],
    )(hcat, bat3, scores, smax, params['Wv'], rb1(params['bv']))

    rW = params['reg_W']
    rb = [rb1(b) for b in params['reg_b']]
    out = pl.pallas_call(
        _final_body,
        out_shape=jax.ShapeDtypeStruct((B, 1), f32),
    )(anum, aden, metal,
      rW[0], rb[0], rW[1], rb[1], rW[2], rb[2], rW[3], rb[3], rW[4], rb[4])
    return jnp.squeeze(out, -1)
